# baseline (speedup 1.0000x reference)
import functools
import jax
import jax.numpy as jnp
from jax import lax
from jax.experimental import pallas as pl
from jax.experimental.pallas import tpu as pltpu
from jax.experimental.pallas import tpu_sc as plsc

_N = 4096
_BA = 256
_IB = 512
_JB = 1024


_E = 131072
_NSUB = 16
_CHUNK = _E // _NSUB
_SW = 256
_NSTRIP = _N // _SW
_ZROWS = 2048
_TE = 128
_K = 4


def _nrm2_body(h_ref, o_ref):
    h = h_ref[...]
    g = lax.dot_general(h, h, (((0,), (0,)), ((), ())),
                        preferred_element_type=jnp.float32)
    hg = jnp.dot(h, g, preferred_element_type=jnp.float32)
    ones = jnp.ones((h.shape[1], 1), jnp.float32)
    o_ref[...] = jnp.dot(hg * h, ones, preferred_element_type=jnp.float32)


def _nrm2_call(h):
    n, di = h.shape
    return pl.pallas_call(
        _nrm2_body,
        grid=(1,),
        in_specs=[pl.BlockSpec((n, di), lambda i: (0, 0))],
        out_specs=pl.BlockSpec((n, 1), lambda i: (0, 0)),
        out_shape=jax.ShapeDtypeStruct((n, 1), jnp.float32),
    )(h)


def _attn_body(*refs, same_prev):
    if same_prev:
        (h_ref, aorig_ref, nrm2_ref, w1_ref, b1_ref, w2_ref,
         am_ref, beta_ref, d_ref) = refs
        aprev_ref = None
    else:
        (h_ref, aprev_ref, aorig_ref, nrm2_ref, w1_ref, b1_ref, w2_ref,
         am_ref, beta_ref, d_ref) = refs
    rb = pl.program_id(0)
    h_blk = h_ref[pl.ds(rb * _BA, _BA), :]
    inner = lax.dot_general(h_blk, h_ref[...], (((1,), (1,)), ((), ())),
                            preferred_element_type=jnp.float32)
    nrm2 = nrm2_ref[pl.ds(rb * _BA, _BA), :]
    scale = 1.0 / jnp.maximum(jnp.sqrt(nrm2), 1e-12)
    aorig = aorig_ref[...]
    aprev = aorig if same_prev else aprev_ref[...]
    rows = rb * _BA + lax.broadcasted_iota(jnp.int32, (_BA, _N), 0)
    cols = lax.broadcasted_iota(jnp.int32, (_BA, _N), 1)
    p = jnp.where(rows != cols, inner * aorig, 0.0)
    b1 = b1_ref[...]
    w2 = w2_ref[...]
    t0 = jnp.tanh(jnp.dot(aprev, w1_ref[...],
                          preferred_element_type=jnp.float32) + b1)
    t1 = jnp.tanh(jnp.dot(p, w1_ref[...],
                          preferred_element_type=jnp.float32) * scale + b1)
    s0 = jnp.sum(t0 * w2, axis=1, keepdims=True)
    s1 = jnp.sum(t1 * w2, axis=1, keepdims=True)
    m = jnp.maximum(s0, s1)
    e0 = jnp.exp(s0 - m)
    e1 = jnp.exp(s1 - m)
    den = e0 + e1
    b0 = e0 / den
    b1s = e1 / den
    am = b0 * aprev + (b1s * scale) * p
    am_ref[...] = am
    ones = jnp.ones((1, _BA), jnp.float32)
    part = jnp.dot(ones, am, preferred_element_type=jnp.float32)

    @pl.when(rb == 0)
    def _():
        d_ref[...] = part

    @pl.when(rb > 0)
    def _():
        d_ref[...] += part

    beta_ref[...] = jnp.concatenate([b0, b1s], axis=1)


def _attn_call(h, aprev, aorig, nrm2, w1, b1, w2):
    h = h.astype(jnp.bfloat16)
    n, di = h.shape
    grid = (n // _BA,)
    same_prev = aprev is None
    specs = [
        pl.BlockSpec((n, di), lambda i: (0, 0)),
        pl.BlockSpec((_BA, n), lambda i: (i, 0)),
        pl.BlockSpec((_BA, n), lambda i: (i, 0)),
        pl.BlockSpec((n, 1), lambda i: (0, 0)),
        pl.BlockSpec((n, 16), lambda i: (0, 0)),
        pl.BlockSpec((1, 16), lambda i: (0, 0)),
        pl.BlockSpec((1, 16), lambda i: (0, 0)),
    ]
    args = [h, aprev, aorig, nrm2, w1, b1, w2]
    if same_prev:
        del specs[1], args[1]
    return pl.pallas_call(
        functools.partial(_attn_body, same_prev=same_prev),
        grid=grid,
        in_specs=specs,
        out_specs=[
            pl.BlockSpec((_BA, n), lambda i: (i, 0)),
            pl.BlockSpec((_BA, 2), lambda i: (i, 0)),
            pl.BlockSpec((1, n), lambda i: (0, 0)),
        ],
        out_shape=[
            jax.ShapeDtypeStruct((n, n), jnp.float32),
            jax.ShapeDtypeStruct((n, 2), jnp.float32),
            jax.ShapeDtypeStruct((1, n), jnp.float32),
        ],
    )(*args)


def _msg_body(h_ref, w_ref, d_ref, msg_ref):
    i = pl.program_id(0)
    dj = d_ref[0, pl.ds(i * _IB, _IB)]
    dinv = jnp.where(dj > 0, lax.rsqrt(dj), 0.0).reshape(_IB, 1)
    msg_ref[...] = dinv * jnp.dot(h_ref[...], w_ref[...],
                                  preferred_element_type=jnp.float32)


def _msg_call(h, w, d):
    n, di = h.shape
    do = w.shape[1]
    return pl.pallas_call(
        _msg_body,
        grid=(n // _IB,),
        in_specs=[
            pl.BlockSpec((_IB, di), lambda i: (i, 0)),
            pl.BlockSpec((di, do), lambda i: (0, 0)),
            pl.BlockSpec((1, n), lambda i: (0, 0)),
        ],
        out_specs=pl.BlockSpec((_IB, do), lambda i: (i, 0)),
        out_shape=jax.ShapeDtypeStruct((n, do), jnp.float32),
    )(h, w, d)


def _agg_body(a_ref, msg_ref, d_ref, b_ref, out_ref, ps_ref, pss_ref):
    j = pl.program_id(0)
    i = pl.program_id(1)
    ni = pl.num_programs(1)
    contrib = lax.dot_general(a_ref[...].astype(jnp.bfloat16),
                              msg_ref[...].astype(jnp.bfloat16),
                              (((0,), (0,)), ((), ())),
                              preferred_element_type=jnp.float32)

    @pl.when(i == 0)
    def _():
        out_ref[...] = contrib

    @pl.when(i > 0)
    def _():
        out_ref[...] += contrib

    @pl.when(i == ni - 1)
    def _():
        dj = d_ref[0, pl.ds(j * _JB, _JB)]
        dinv = jnp.where(dj > 0, lax.rsqrt(dj), 0.0).reshape(_JB, 1)
        val = out_ref[...] * dinv + b_ref[...]
        out_ref[...] = val
        ps_ref[...] = jnp.sum(val, axis=0).reshape(1, 1, -1)
        pss_ref[...] = jnp.sum(val * val, axis=0).reshape(1, 1, -1)


def _agg_call(am, msg, d, bias):
    n = am.shape[0]
    do = msg.shape[1]
    nj = n // _JB
    ni = n // _IB
    return pl.pallas_call(
        _agg_body,
        grid=(nj, ni),
        in_specs=[
            pl.BlockSpec((_IB, _JB), lambda j, i: (i, j)),
            pl.BlockSpec((_IB, do), lambda j, i: (i, 0)),
            pl.BlockSpec((1, n), lambda j, i: (0, 0)),
            pl.BlockSpec((1, do), lambda j, i: (0, 0)),
        ],
        out_specs=[
            pl.BlockSpec((_JB, do), lambda j, i: (j, 0)),
            pl.BlockSpec((1, 1, do), lambda j, i: (j, 0, 0)),
            pl.BlockSpec((1, 1, do), lambda j, i: (j, 0, 0)),
        ],
        out_shape=[
            jax.ShapeDtypeStruct((n, do), jnp.float32),
            jax.ShapeDtypeStruct((nj, 1, do), jnp.float32),
            jax.ShapeDtypeStruct((nj, 1, do), jnp.float32),
        ],
    )(am, msg, d, bias)


def _bn_body(*refs, nf, head):
    if head:
        (x_ref, ps_ref, pss_ref, g_ref, b_ref, lw_ref, lb_ref,
         o_ref, out_ref, acc_ref) = refs
    else:
        x_ref, ps_ref, pss_ref, g_ref, b_ref, o_ref = refs
    s = jnp.sum(ps_ref[...], axis=(0, 1)).reshape(1, -1)
    ss = jnp.sum(pss_ref[...], axis=(0, 1)).reshape(1, -1)
    mu = s / nf
    var = ss / nf - mu * mu
    rstd = lax.rsqrt(var + 1e-5)
    y = (x_ref[...] - mu) * rstd * g_ref[...] + b_ref[...]
    y = jnp.where(y >= 0, y, 0.01 * y)
    o_ref[...] = y
    if head:
        i = pl.program_id(0)
        ones = jnp.ones((1, y.shape[0]), jnp.float32)
        part = jnp.dot(ones, y, preferred_element_type=jnp.float32)

        @pl.when(i == 0)
        def _():
            acc_ref[...] = part

        @pl.when(i > 0)
        def _():
            acc_ref[...] += part

        @pl.when(i == pl.num_programs(0) - 1)
        def _():
            pooled = acc_ref[...] / nf
            logits = jnp.dot(pooled, lw_ref[...],
                             preferred_element_type=jnp.float32) + lb_ref[...]
            m = jnp.max(logits, axis=1, keepdims=True)
            e = jnp.exp(logits - m)
            out_ref[...] = e / jnp.sum(e, axis=1, keepdims=True)


def _bn_call(x, ps, pss, g, b, lw=None, lb=None):
    n, do = x.shape
    nj = ps.shape[0]
    head = lw is not None
    specs = [
        pl.BlockSpec((_IB, do), lambda i: (i, 0)),
        pl.BlockSpec((nj, 1, do), lambda i: (0, 0, 0)),
        pl.BlockSpec((nj, 1, do), lambda i: (0, 0, 0)),
        pl.BlockSpec((1, do), lambda i: (0, 0)),
        pl.BlockSpec((1, do), lambda i: (0, 0)),
    ]
    args = [x, ps, pss, g, b]
    out_specs = [pl.BlockSpec((_IB, do), lambda i: (i, 0))]
    out_shape = [jax.ShapeDtypeStruct((n, do), jnp.float32)]
    scratch = []
    if head:
        dh = lw.shape[1]
        specs += [pl.BlockSpec((do, dh), lambda i: (0, 0)),
                  pl.BlockSpec((1, dh), lambda i: (0, 0))]
        args += [lw, lb]
        out_specs.append(pl.BlockSpec((1, dh), lambda i: (0, 0)))
        out_shape.append(jax.ShapeDtypeStruct((1, dh), jnp.float32))
        scratch = [pltpu.VMEM((1, do), jnp.float32)]
    res = pl.pallas_call(
        functools.partial(_bn_body, nf=float(n), head=head),
        grid=(n // _IB,),
        in_specs=specs,
        out_specs=out_specs,
        out_shape=out_shape,
        scratch_shapes=scratch,
    )(*args)
    return res if head else res[0]


def kernel(X, A, W, batch,
           attW1_0, attb1_0, attW2_0, attW1_1, attb1_1, attW2_1,
           gcnW0, gcnb0, gcnW1, gcnb1,
           bng0, bnb0, bng1, bnb1, linW, linb):
    n = X.shape[0]
    a_orig = jnp.zeros((n, n), jnp.float32).at[A[0], A[1]].add(W)

    h = X
    aprev = None
    am = beta = None
    atts = [(attW1_0, attb1_0, attW2_0), (attW1_1, attb1_1, attW2_1)]
    gcns = [(gcnW0, gcnb0), (gcnW1, gcnb1)]
    bns = [(bng0, bnb0), (bng1, bnb1)]
    for i in range(2):
        w1, b1, w2 = atts[i]
        nrm2 = _nrm2_call(h)
        am, beta, d = _attn_call(h, aprev, a_orig, nrm2, w1,
                                 b1.reshape(1, -1), w2.reshape(1, -1))
        gw, gb = gcns[i]
        msg = _msg_call(h, gw, d)
        hpre, ps, pss = _agg_call(am, msg, d, gb.reshape(1, -1))
        g, b = bns[i]
        if i == 1:
            h, out = _bn_call(hpre, ps, pss, g.reshape(1, -1),
                              b.reshape(1, -1), linW, linb.reshape(1, -1))
        else:
            h = _bn_call(hpre, ps, pss, g.reshape(1, -1), b.reshape(1, -1))
        aprev = am

    return out, h, am, beta.reshape(n, 2, 1)


# speedup vs baseline: 1.0051x; 1.0051x over previous
import functools
import jax
import jax.numpy as jnp
from jax import lax
from jax.experimental import pallas as pl
from jax.experimental.pallas import tpu as pltpu

_N = 4096
_BA = 256
_IB = 512
_JB = 1024


def _nrm2_body(h_ref, o_ref):
    h = h_ref[...]
    g = lax.dot_general(h, h, (((0,), (0,)), ((), ())),
                        preferred_element_type=jnp.float32)
    hg = jnp.dot(h, g, preferred_element_type=jnp.float32)
    ones = jnp.ones((h.shape[1], 1), jnp.float32)
    o_ref[...] = jnp.dot(hg * h, ones, preferred_element_type=jnp.float32)


def _nrm2_call(h):
    n, di = h.shape
    return pl.pallas_call(
        _nrm2_body,
        grid=(1,),
        in_specs=[pl.BlockSpec((n, di), lambda i: (0, 0))],
        out_specs=pl.BlockSpec((n, 1), lambda i: (0, 0)),
        out_shape=jax.ShapeDtypeStruct((n, 1), jnp.float32),
    )(h)


def _attn_body(*refs, same_prev):
    if same_prev:
        (h_ref, aorig_ref, nrm2_ref, w1_ref, b1_ref, w2_ref,
         am_ref, beta_ref, d_ref) = refs
        aprev_ref = None
    else:
        (h_ref, aprev_ref, aorig_ref, nrm2_ref, w1_ref, b1_ref, w2_ref,
         am_ref, beta_ref, d_ref) = refs
    rb = pl.program_id(0)
    h_blk = h_ref[pl.ds(rb * _BA, _BA), :]
    inner = lax.dot_general(h_blk, h_ref[...], (((1,), (1,)), ((), ())),
                            preferred_element_type=jnp.float32)
    nrm2 = nrm2_ref[pl.ds(rb * _BA, _BA), :]
    scale = 1.0 / jnp.maximum(jnp.sqrt(nrm2), 1e-12)
    aorig = aorig_ref[...]
    aprev = aorig if same_prev else aprev_ref[...]
    rows = rb * _BA + lax.broadcasted_iota(jnp.int32, (_BA, _N), 0)
    cols = lax.broadcasted_iota(jnp.int32, (_BA, _N), 1)
    p = jnp.where(rows != cols, inner * aorig, 0.0)
    b1 = b1_ref[...]
    w2 = w2_ref[...]
    t0 = jnp.tanh(jnp.dot(aprev, w1_ref[...],
                          preferred_element_type=jnp.float32) + b1)
    t1 = jnp.tanh(jnp.dot(p, w1_ref[...],
                          preferred_element_type=jnp.float32) * scale + b1)
    s0 = jnp.sum(t0 * w2, axis=1, keepdims=True)
    s1 = jnp.sum(t1 * w2, axis=1, keepdims=True)
    m = jnp.maximum(s0, s1)
    e0 = jnp.exp(s0 - m)
    e1 = jnp.exp(s1 - m)
    den = e0 + e1
    b0 = e0 / den
    b1s = e1 / den
    am = b0 * aprev + (b1s * scale) * p
    am_ref[...] = am
    ones = jnp.ones((1, _BA), jnp.float32)
    part = jnp.dot(ones, am, preferred_element_type=jnp.float32)

    @pl.when(rb == 0)
    def _():
        d_ref[...] = part

    @pl.when(rb > 0)
    def _():
        d_ref[...] += part

    beta_ref[...] = jnp.concatenate([b0, b1s], axis=1)


def _attn_call(h, aprev, aorig, nrm2, w1, b1, w2):
    n, di = h.shape
    grid = (n // _BA,)
    same_prev = aprev is None
    specs = [
        pl.BlockSpec((n, di), lambda i: (0, 0)),
        pl.BlockSpec((_BA, n), lambda i: (i, 0)),
        pl.BlockSpec((_BA, n), lambda i: (i, 0)),
        pl.BlockSpec((n, 1), lambda i: (0, 0)),
        pl.BlockSpec((n, 16), lambda i: (0, 0)),
        pl.BlockSpec((1, 16), lambda i: (0, 0)),
        pl.BlockSpec((1, 16), lambda i: (0, 0)),
    ]
    args = [h, aprev, aorig, nrm2, w1, b1, w2]
    if same_prev:
        del specs[1], args[1]
    return pl.pallas_call(
        functools.partial(_attn_body, same_prev=same_prev),
        grid=grid,
        in_specs=specs,
        out_specs=[
            pl.BlockSpec((_BA, n), lambda i: (i, 0)),
            pl.BlockSpec((_BA, 2), lambda i: (i, 0)),
            pl.BlockSpec((1, n), lambda i: (0, 0)),
        ],
        out_shape=[
            jax.ShapeDtypeStruct((n, n), jnp.float32),
            jax.ShapeDtypeStruct((n, 2), jnp.float32),
            jax.ShapeDtypeStruct((1, n), jnp.float32),
        ],
    )(*args)


def _msg_body(h_ref, w_ref, d_ref, msg_ref):
    i = pl.program_id(0)
    dj = d_ref[0, pl.ds(i * _IB, _IB)]
    dinv = jnp.where(dj > 0, lax.rsqrt(dj), 0.0).reshape(_IB, 1)
    msg_ref[...] = dinv * jnp.dot(h_ref[...], w_ref[...],
                                  preferred_element_type=jnp.float32)


def _msg_call(h, w, d):
    n, di = h.shape
    do = w.shape[1]
    return pl.pallas_call(
        _msg_body,
        grid=(n // _IB,),
        in_specs=[
            pl.BlockSpec((_IB, di), lambda i: (i, 0)),
            pl.BlockSpec((di, do), lambda i: (0, 0)),
            pl.BlockSpec((1, n), lambda i: (0, 0)),
        ],
        out_specs=pl.BlockSpec((_IB, do), lambda i: (i, 0)),
        out_shape=jax.ShapeDtypeStruct((n, do), jnp.float32),
    )(h, w, d)


def _agg_body(a_ref, msg_ref, d_ref, b_ref, out_ref, ps_ref, pss_ref):
    j = pl.program_id(0)
    i = pl.program_id(1)
    ni = pl.num_programs(1)
    contrib = lax.dot_general(a_ref[...].astype(jnp.bfloat16),
                              msg_ref[...].astype(jnp.bfloat16),
                              (((0,), (0,)), ((), ())),
                              preferred_element_type=jnp.float32)

    @pl.when(i == 0)
    def _():
        out_ref[...] = contrib

    @pl.when(i > 0)
    def _():
        out_ref[...] += contrib

    @pl.when(i == ni - 1)
    def _():
        dj = d_ref[0, pl.ds(j * _JB, _JB)]
        dinv = jnp.where(dj > 0, lax.rsqrt(dj), 0.0).reshape(_JB, 1)
        val = out_ref[...] * dinv + b_ref[...]
        out_ref[...] = val
        ps_ref[...] = jnp.sum(val, axis=0).reshape(1, 1, -1)
        pss_ref[...] = jnp.sum(val * val, axis=0).reshape(1, 1, -1)


def _agg_call(am, msg, d, bias):
    n = am.shape[0]
    do = msg.shape[1]
    nj = n // _JB
    ni = n // _IB
    return pl.pallas_call(
        _agg_body,
        grid=(nj, ni),
        in_specs=[
            pl.BlockSpec((_IB, _JB), lambda j, i: (i, j)),
            pl.BlockSpec((_IB, do), lambda j, i: (i, 0)),
            pl.BlockSpec((1, n), lambda j, i: (0, 0)),
            pl.BlockSpec((1, do), lambda j, i: (0, 0)),
        ],
        out_specs=[
            pl.BlockSpec((_JB, do), lambda j, i: (j, 0)),
            pl.BlockSpec((1, 1, do), lambda j, i: (j, 0, 0)),
            pl.BlockSpec((1, 1, do), lambda j, i: (j, 0, 0)),
        ],
        out_shape=[
            jax.ShapeDtypeStruct((n, do), jnp.float32),
            jax.ShapeDtypeStruct((nj, 1, do), jnp.float32),
            jax.ShapeDtypeStruct((nj, 1, do), jnp.float32),
        ],
    )(am, msg, d, bias)


def _bn_body(*refs, nf, head):
    if head:
        (x_ref, ps_ref, pss_ref, g_ref, b_ref, lw_ref, lb_ref,
         o_ref, out_ref, acc_ref) = refs
    else:
        x_ref, ps_ref, pss_ref, g_ref, b_ref, o_ref = refs
    s = jnp.sum(ps_ref[...], axis=(0, 1)).reshape(1, -1)
    ss = jnp.sum(pss_ref[...], axis=(0, 1)).reshape(1, -1)
    mu = s / nf
    var = ss / nf - mu * mu
    rstd = lax.rsqrt(var + 1e-5)
    y = (x_ref[...] - mu) * rstd * g_ref[...] + b_ref[...]
    y = jnp.where(y >= 0, y, 0.01 * y)
    o_ref[...] = y
    if head:
        i = pl.program_id(0)
        ones = jnp.ones((1, y.shape[0]), jnp.float32)
        part = jnp.dot(ones, y, preferred_element_type=jnp.float32)

        @pl.when(i == 0)
        def _():
            acc_ref[...] = part

        @pl.when(i > 0)
        def _():
            acc_ref[...] += part

        @pl.when(i == pl.num_programs(0) - 1)
        def _():
            pooled = acc_ref[...] / nf
            logits = jnp.dot(pooled, lw_ref[...],
                             preferred_element_type=jnp.float32) + lb_ref[...]
            m = jnp.max(logits, axis=1, keepdims=True)
            e = jnp.exp(logits - m)
            out_ref[...] = e / jnp.sum(e, axis=1, keepdims=True)


def _bn_call(x, ps, pss, g, b, lw=None, lb=None):
    n, do = x.shape
    nj = ps.shape[0]
    head = lw is not None
    specs = [
        pl.BlockSpec((_IB, do), lambda i: (i, 0)),
        pl.BlockSpec((nj, 1, do), lambda i: (0, 0, 0)),
        pl.BlockSpec((nj, 1, do), lambda i: (0, 0, 0)),
        pl.BlockSpec((1, do), lambda i: (0, 0)),
        pl.BlockSpec((1, do), lambda i: (0, 0)),
    ]
    args = [x, ps, pss, g, b]
    out_specs = [pl.BlockSpec((_IB, do), lambda i: (i, 0))]
    out_shape = [jax.ShapeDtypeStruct((n, do), jnp.float32)]
    scratch = []
    if head:
        dh = lw.shape[1]
        specs += [pl.BlockSpec((do, dh), lambda i: (0, 0)),
                  pl.BlockSpec((1, dh), lambda i: (0, 0))]
        args += [lw, lb]
        out_specs.append(pl.BlockSpec((1, dh), lambda i: (0, 0)))
        out_shape.append(jax.ShapeDtypeStruct((1, dh), jnp.float32))
        scratch = [pltpu.VMEM((1, do), jnp.float32)]
    res = pl.pallas_call(
        functools.partial(_bn_body, nf=float(n), head=head),
        grid=(n // _IB,),
        in_specs=specs,
        out_specs=out_specs,
        out_shape=out_shape,
        scratch_shapes=scratch,
    )(*args)
    return res if head else res[0]


def kernel(X, A, W, batch,
           attW1_0, attb1_0, attW2_0, attW1_1, attb1_1, attW2_1,
           gcnW0, gcnb0, gcnW1, gcnb1,
           bng0, bnb0, bng1, bnb1, linW, linb):
    n = X.shape[0]
    a_orig = jnp.zeros((n, n), jnp.float32).at[A[0], A[1]].add(W)

    h = X
    aprev = None
    am = beta = None
    atts = [(attW1_0, attb1_0, attW2_0), (attW1_1, attb1_1, attW2_1)]
    gcns = [(gcnW0, gcnb0), (gcnW1, gcnb1)]
    bns = [(bng0, bnb0), (bng1, bnb1)]
    for i in range(2):
        w1, b1, w2 = atts[i]
        nrm2 = _nrm2_call(h)
        am, beta, d = _attn_call(h, aprev, a_orig, nrm2, w1,
                                 b1.reshape(1, -1), w2.reshape(1, -1))
        gw, gb = gcns[i]
        msg = _msg_call(h, gw, d)
        hpre, ps, pss = _agg_call(am, msg, d, gb.reshape(1, -1))
        g, b = bns[i]
        if i == 1:
            h, out = _bn_call(hpre, ps, pss, g.reshape(1, -1),
                              b.reshape(1, -1), linW, linb.reshape(1, -1))
        else:
            h = _bn_call(hpre, ps, pss, g.reshape(1, -1), b.reshape(1, -1))
        aprev = am

    return out, h, am, beta.reshape(n, 2, 1)
